# Initial kernel scaffold; baseline (speedup 1.0000x reference)
import jax
import jax.numpy as jnp
from jax.experimental import pallas as pl


def kernel(inputs, token_table, pos_table):
    raise NotImplementedError("write your pallas kernel here")


# speedup vs baseline: 1.2968x; 1.2968x over previous
import functools

import jax
import jax.numpy as jnp
from jax import lax
from jax.experimental import pallas as pl
from jax.experimental.pallas import tpu as pltpu
from jax.experimental.pallas import tpu_sc as plsc

_SEQ = 200
_BATCH = 4096
_DIM = 32
_NC = 2
_NS = 16
_NW = _NC * _NS
_SEQ_PER_W = _BATCH // _NW
_ROWS_PER_W = _SEQ_PER_W * _SEQ


def _body(idx_hbm, tok_hbm, pos_hbm, out_hbm, idx_v, pos_v, buf, sem):
    c = lax.axis_index("c")
    s = lax.axis_index("s")
    wid = s * _NC + c
    base = wid * _ROWS_PER_W

    pltpu.sync_copy(idx_hbm.at[pl.ds(base, _ROWS_PER_W)], idx_v)
    pltpu.sync_copy(pos_hbm, pos_v)

    def step(t, carry):
        off = t * _SEQ
        h1 = pltpu.async_copy(
            tok_hbm.at[idx_v.at[pl.ds(off, 128)]], buf.at[pl.ds(0, 128)], sem)
        h2 = pltpu.async_copy(
            tok_hbm.at[idx_v.at[pl.ds(off + 128, 72)]], buf.at[pl.ds(128, 72)], sem)
        h1.wait()
        h2.wait()

        def add8(i, carry2):
            r0 = i * 8
            for j in range(8):
                r = r0 + j
                for cix in (0, 16):
                    pv = pos_v[r, pl.ds(cix, 16)]
                    plsc.addupdate(buf.at[r, pl.ds(cix, 16)], pv)
            return carry2

        lax.fori_loop(0, _SEQ // 8, add8, 0, unroll=False)
        pltpu.sync_copy(buf, out_hbm.at[pl.ds(base + off, _SEQ)])
        return carry

    lax.fori_loop(0, _SEQ_PER_W, step, 0, unroll=False)


@jax.jit
def kernel(inputs, token_table, pos_table):
    idx = inputs.reshape(-1).astype(jnp.int32)
    run = pl.kernel(
        _body,
        out_type=jax.ShapeDtypeStruct((_BATCH * _SEQ, _DIM), jnp.float32),
        mesh=plsc.VectorSubcoreMesh(core_axis_name="c", subcore_axis_name="s"),
        compiler_params=pltpu.CompilerParams(use_tc_tiling_on_sc=False),
        scratch_types=[
            pltpu.VMEM((_ROWS_PER_W,), jnp.int32),
            pltpu.VMEM((_SEQ, _DIM), jnp.float32),
            pltpu.VMEM((_SEQ, _DIM), jnp.float32),
            pltpu.SemaphoreType.DMA,
        ],
    )
    out = run(idx, token_table, pos_table)
    return out.reshape(_BATCH, _SEQ, _DIM)


# speedup vs baseline: 1.4477x; 1.1164x over previous
import functools

import jax
import jax.numpy as jnp
from jax import lax
from jax.experimental import pallas as pl
from jax.experimental.pallas import tpu as pltpu
from jax.experimental.pallas import tpu_sc as plsc

_SEQ = 200
_BATCH = 4096
_DIM = 32
_NC = 2
_NS = 16
_NW = _NC * _NS
_SEQ_PER_W = _BATCH // _NW
_ROWS_PER_W = _SEQ_PER_W * _SEQ
_NBUF = 4
_PREF = 2


def _body(idx_hbm, tok_hbm, pos_hbm, out_hbm, idx_v, pos_v, bufs, gsems, osems):
    c = lax.axis_index("c")
    s = lax.axis_index("s")
    wid = s * _NC + c
    base = wid * _ROWS_PER_W

    pltpu.sync_copy(idx_hbm.at[pl.ds(base, _ROWS_PER_W)], idx_v)
    pltpu.sync_copy(pos_hbm, pos_v)

    def start_gather(t, b):
        off = t * _SEQ
        pltpu.async_copy(
            tok_hbm.at[idx_v.at[pl.ds(off, 128)]],
            bufs.at[b, pl.ds(0, 128)], gsems.at[b])
        pltpu.async_copy(
            tok_hbm.at[idx_v.at[pl.ds(off + 128, 72)]],
            bufs.at[b, pl.ds(128, 72)], gsems.at[b])

    def wait_gather(b):
        pltpu.make_async_copy(
            tok_hbm.at[pl.ds(0, 128)], bufs.at[b, pl.ds(0, 128)],
            gsems.at[b]).wait()
        pltpu.make_async_copy(
            tok_hbm.at[pl.ds(0, 72)], bufs.at[b, pl.ds(128, 72)],
            gsems.at[b]).wait()

    def start_out(t, b):
        pltpu.async_copy(
            bufs.at[b], out_hbm.at[pl.ds(base + t * _SEQ, _SEQ)], osems.at[b])

    def wait_out(b):
        pltpu.make_async_copy(
            bufs.at[b], out_hbm.at[pl.ds(0, _SEQ)], osems.at[b]).wait()

    def add_pos(b):
        def add8(i, carry):
            r0 = i * 8
            for j in range(8):
                r = r0 + j
                for cix in (0, 16):
                    pv = pos_v[r, pl.ds(cix, 16)]
                    plsc.addupdate(bufs.at[b, r, pl.ds(cix, 16)], pv)
            return carry
        lax.fori_loop(0, _SEQ // 8, add8, 0, unroll=False)

    def slot(t, b, first_group, last_group):
        wait_gather(b)
        add_pos(b)
        start_out(t, b)
        if not last_group:
            b2 = (b + _PREF) % _NBUF
            if not first_group:
                wait_out(b2)
            start_gather(t + _PREF, b2)

    for b in range(_PREF):
        start_gather(b, b)

    for b in range(_NBUF):
        if b < _PREF:
            slot(b, b, first_group=True, last_group=False)
        else:
            slot(b, b, first_group=False, last_group=False)

    def group(g, carry):
        for b in range(_NBUF):
            slot(g * _NBUF + b, b, first_group=False, last_group=False)
        return carry

    lax.fori_loop(1, _SEQ_PER_W // _NBUF - 1, group, 0, unroll=False)

    g_last = _SEQ_PER_W // _NBUF - 1
    for b in range(_NBUF):
        t = g_last * _NBUF + b
        if t + _PREF < _SEQ_PER_W:
            slot(t, b, first_group=False, last_group=False)
        else:
            slot(t, b, first_group=False, last_group=True)

    for b in range(_NBUF):
        wait_out(b)


@jax.jit
def kernel(inputs, token_table, pos_table):
    idx = inputs.reshape(-1).astype(jnp.int32)
    run = pl.kernel(
        _body,
        out_type=jax.ShapeDtypeStruct((_BATCH * _SEQ, _DIM), jnp.float32),
        mesh=plsc.VectorSubcoreMesh(core_axis_name="c", subcore_axis_name="s"),
        compiler_params=pltpu.CompilerParams(use_tc_tiling_on_sc=False),
        scratch_types=[
            pltpu.VMEM((_ROWS_PER_W,), jnp.int32),
            pltpu.VMEM((_SEQ, _DIM), jnp.float32),
            pltpu.VMEM((_NBUF, _SEQ, _DIM), jnp.float32),
            pltpu.SemaphoreType.DMA((_NBUF,)),
            pltpu.SemaphoreType.DMA((_NBUF,)),
        ],
    )
    out = run(idx, token_table, pos_table)
    return out.reshape(_BATCH, _SEQ, _DIM)
